# Initial kernel scaffold; baseline (speedup 1.0000x reference)
import jax
import jax.numpy as jnp
from jax.experimental import pallas as pl


def kernel(x, edge_index, W1, b1, W2, b2, gamma, beta, eps):
    raise NotImplementedError("write your pallas kernel here")


# speedup vs baseline: 4.8641x; 4.8641x over previous
import functools

import jax
import jax.numpy as jnp
from jax import lax
from jax.experimental import pallas as pl
from jax.experimental.pallas import tpu as pltpu
from jax.experimental.pallas import tpu_sc as plsc

N = 10000
E = 320000
D = 128
BN_EPS = 1e-5

NC = 2
NS = 16
NW = NC * NS
CHUNK = 128
EPT = ((E // NW + CHUNK - 1) // CHUNK) * CHUNK
NCH = EPT // CHUNK
EP = EPT * NW
ACC_ROWS = ((N + NS - 1) // NS + 39) // 40 * 40 * NS
ROWS_PER_TILE = ACC_ROWS // NS

def _sc_body(x_hbm, src_hbm, dst_hbm, zeros_hbm, out_hbm,
             src_idx, dst_idx, rows, acc, sem):
    cid = lax.axis_index("c")
    sid = lax.axis_index("s")
    wid = cid * NS + sid

    stripe = pl.ds(sid * ROWS_PER_TILE, ROWS_PER_TILE)
    pltpu.sync_copy(zeros_hbm.at[stripe], acc.at[stripe])

    pltpu.sync_copy(src_hbm.at[wid], src_idx)
    pltpu.sync_copy(dst_hbm.at[wid], dst_idx)

    plsc.subcore_barrier()

    def step(j, carry):
        pltpu.async_copy(x_hbm.at[src_idx.at[j]], rows, sem).wait()
        pltpu.sync_copy(rows, acc.at[dst_idx.at[j]], add=True)
        return carry

    lax.fori_loop(0, NCH, step, 0)

    plsc.subcore_barrier()

    pltpu.sync_copy(acc.at[stripe], out_hbm.at[cid].at[stripe])


@functools.cache
def _sc_segment_sum():
    mesh = plsc.VectorSubcoreMesh(
        core_axis_name="c", subcore_axis_name="s",
        num_cores=NC, num_subcores=NS)
    return pl.kernel(
        _sc_body,
        out_type=jax.ShapeDtypeStruct((NC, ACC_ROWS, D), jnp.float32),
        mesh=mesh,
        scratch_types=[
            pltpu.VMEM((NCH, CHUNK), jnp.int32),
            pltpu.VMEM((NCH, CHUNK), jnp.int32),
            pltpu.VMEM((CHUNK, D), jnp.float32),
            pltpu.VMEM_SHARED((ACC_ROWS, D), jnp.float32),
            pltpu.SemaphoreType.DMA,
        ],
    )


_BLK = 1000
_GRID = N // _BLK


def _tc_mlp_body(eps_ref, x_ref, n0_ref, n1_ref, w1_ref, b1_ref, w2_ref,
                 b2_ref, h2_ref, stats_ref):
    i = pl.program_id(0)
    eps = eps_ref[0]
    m = (1.0 + eps) * x_ref[...] + n0_ref[...] + n1_ref[...]
    a1 = jnp.maximum(
        jnp.dot(m, w1_ref[...], preferred_element_type=jnp.float32)
        + b1_ref[...], 0.0)
    h2 = (jnp.dot(a1, w2_ref[...], preferred_element_type=jnp.float32)
          + b2_ref[...])
    h2_ref[...] = h2
    s1 = jnp.sum(h2, axis=0, keepdims=True)
    s2 = jnp.sum(h2 * h2, axis=0, keepdims=True)
    blk = jnp.concatenate([s1, s2, jnp.zeros((6, D), jnp.float32)], axis=0)

    @pl.when(i == 0)
    def _():
        stats_ref[...] = blk

    @pl.when(i > 0)
    def _():
        stats_ref[...] += blk


def _tc_bn_body(h2_ref, x_ref, stats_ref, g_ref, b_ref, out_ref):
    mean = stats_ref[0:1, :] / N
    var = stats_ref[1:2, :] / N - mean * mean
    inv = lax.rsqrt(var + BN_EPS)
    h = g_ref[...] * (h2_ref[...] - mean) * inv + b_ref[...]
    out_ref[...] = x_ref[...] + jnp.maximum(h, 0.0)


def kernel(x, edge_index, W1, b1, W2, b2, gamma, beta, eps):
    src = edge_index[0]
    dst = edge_index[1]
    pad = EP - E
    src_p = jnp.concatenate(
        [src, jnp.zeros((pad,), jnp.int32)]).reshape(NW, NCH, CHUNK)
    dst_p = jnp.concatenate(
        [dst, jnp.full((pad,), N, jnp.int32)]).reshape(NW, NCH, CHUNK)
    zeros = jnp.zeros((ACC_ROWS, D), jnp.float32)

    nacc = _sc_segment_sum()(x, src_p, dst_p, zeros)

    row_spec = pl.BlockSpec((_BLK, D), lambda i: (i, 0))
    full_mat = pl.BlockSpec((D, D), lambda i: (0, 0))
    full_vec = pl.BlockSpec((1, D), lambda i: (0, 0))
    stat_spec = pl.BlockSpec((8, D), lambda i: (0, 0))

    h2, stats = pl.pallas_call(
        _tc_mlp_body,
        grid=(_GRID,),
        in_specs=[
            pl.BlockSpec(memory_space=pltpu.SMEM),
            row_spec, row_spec, row_spec,
            full_mat, full_vec, full_mat, full_vec,
        ],
        out_specs=[row_spec, stat_spec],
        out_shape=[
            jax.ShapeDtypeStruct((N, D), jnp.float32),
            jax.ShapeDtypeStruct((8, D), jnp.float32),
        ],
    )(eps.reshape(1), x, nacc[0], nacc[1], W1, b1.reshape(1, D),
      W2, b2.reshape(1, D))

    out = pl.pallas_call(
        _tc_bn_body,
        grid=(_GRID,),
        in_specs=[row_spec, row_spec, stat_spec, full_vec, full_vec],
        out_specs=row_spec,
        out_shape=jax.ShapeDtypeStruct((N, D), jnp.float32),
    )(h2, x, stats, gamma.reshape(1, D), beta.reshape(1, D))

    return out
